# baseline (speedup 1.0000x reference)
import functools

import jax
import jax.numpy as jnp
from jax import lax
from jax.experimental import pallas as pl
from jax.experimental.pallas import tpu as pltpu
from jax.experimental.pallas import tpu_sc as plsc

N_ROWS = 16384
N_FACT = 32
LANES = 16

_info = plsc.get_sparse_core_info()
_NC, _NS = _info.num_cores, _info.num_subcores
_NW = _NC * _NS
_BPW = N_ROWS // _NW
_CHUNK = 128
_NCH = _BPW // _CHUNK
_IDX_ROWS = N_ROWS // _CHUNK


def _sc_body(x0_hbm, x1_hbm, ct_hbm, pt_hbm, wb_hbm, out_hbm,
             idx_c, idx_p, c_cols, p_cols, wb_v, out_v, sem):
    cid = lax.axis_index("c")
    sid = lax.axis_index("s")
    wid = sid * _NC + cid

    pltpu.sync_copy(x0_hbm.at[pl.ds(wid * _NCH, _NCH)], idx_c)
    pltpu.sync_copy(x1_hbm.at[pl.ds(wid * _NCH, _NCH)], idx_p)
    pltpu.sync_copy(wb_hbm, wb_v)

    for k0 in range(0, N_FACT, 2):
        cps = []
        for k in (k0, k0 + 1):
            for j in range(_NCH):
                cps.append(pltpu.async_copy(
                    ct_hbm.at[k].at[idx_c.at[j]],
                    c_cols.at[k, pl.ds(j * _CHUNK, _CHUNK)], sem))
                cps.append(pltpu.async_copy(
                    pt_hbm.at[k].at[idx_p.at[j]],
                    p_cols.at[k, pl.ds(j * _CHUNK, _CHUNK)], sem))
        for cp in cps:
            cp.wait()

    w = wb_v[0, :]
    b = wb_v[1, :]

    def grp(g, carry):
        acc = jnp.zeros((LANES,), jnp.float32)
        for k in range(N_FACT):
            cv = c_cols[k, pl.ds(g * LANES, LANES)]
            pv = p_cols[k, pl.ds(g * LANES, LANES)]
            acc = acc + cv * pv
        y = acc * w + b
        out_v[pl.ds(g * LANES, LANES)] = 1.0 / (1.0 + jnp.exp(-y))
        return carry

    lax.fori_loop(0, _BPW // LANES, grp, 0)
    pltpu.sync_copy(out_v, out_hbm.at[pl.ds(wid * _BPW, _BPW)])


_sc_call = functools.partial(
    pl.kernel,
    mesh=plsc.VectorSubcoreMesh(core_axis_name="c", subcore_axis_name="s"),
    out_type=jax.ShapeDtypeStruct((N_ROWS,), jnp.float32),
    compiler_params=pltpu.CompilerParams(
        needs_layout_passes=False, use_tc_tiling_on_sc=False),
    scratch_types=[
        pltpu.VMEM((_NCH, _CHUNK), jnp.int32),
        pltpu.VMEM((_NCH, _CHUNK), jnp.int32),
        pltpu.VMEM((N_FACT, _BPW), jnp.float32),
        pltpu.VMEM((N_FACT, _BPW), jnp.float32),
        pltpu.VMEM((2, LANES), jnp.float32),
        pltpu.VMEM((_BPW,), jnp.float32),
        pltpu.SemaphoreType.DMA,
    ],
)(_sc_body)


def kernel(x, cust_embedding, prod_embedding, out_w, out_b):
    x0 = x[:, 0].reshape(_IDX_ROWS, _CHUNK)
    x1 = x[:, 1].reshape(_IDX_ROWS, _CHUNK)
    wb = jnp.stack([
        jnp.full((LANES,), out_w[0, 0], jnp.float32),
        jnp.full((LANES,), out_b[0], jnp.float32),
    ])
    out = _sc_call(x0, x1, cust_embedding.T, prod_embedding.T, wb)
    return out.reshape(N_ROWS, 1)


# speedup vs baseline: 19.9292x; 19.9292x over previous
import functools

import jax
import jax.numpy as jnp
from jax import lax
from jax.experimental import pallas as pl
from jax.experimental.pallas import tpu as pltpu
from jax.experimental.pallas import tpu_sc as plsc

N_ROWS = 16384
N_FACT = 32
LANES = 16
VOCAB = 1000000

_info = plsc.get_sparse_core_info()
_NC, _NS = _info.num_cores, _info.num_subcores
_NW = _NC * _NS
_BPW = N_ROWS // _NW
_CHUNK = 128
_NCH = _BPW // _CHUNK
_IDX_ROWS = N_ROWS // _CHUNK

_KB = N_FACT // 8
_NT = (VOCAB + _CHUNK - 1) // _CHUNK
_W = 512
_TPW = _W // _CHUNK
_NWIN = (_NT - 1) // _TPW
_FLAT = _KB * _NT * 8 * _CHUNK


def _unpack_one(t_hbm, out4, wbuf, rsem, wsem, wid, nw):
    def read_win(j, b):
        c0 = pl.multiple_of((j * _NW + wid) * _W, _W)
        return pltpu.async_copy(t_hbm.at[:, pl.ds(c0, _W)], wbuf.at[b], rsem)

    def write_tiles(j, b):
        t0 = (j * _NW + wid) * _TPW
        for kb in range(_KB):
            for c in range(_TPW):
                pltpu.async_copy(
                    wbuf.at[b, pl.ds(kb * 8, 8), pl.ds(c * _CHUNK, _CHUNK)],
                    out4.at[kb, t0 + c], wsem)

    def drain_writes(b):
        for _ in range(_KB * _TPW):
            pltpu.make_async_copy(
                wbuf.at[b, pl.ds(0, 8), pl.ds(0, _CHUNK)],
                out4.at[0, 0], wsem).wait()

    read_win(0, 0)

    def step(j, carry):
        b = lax.rem(j, 2)

        @pl.when(j >= 1)
        def _():
            drain_writes(1 - b)

        @pl.when(j + 1 < nw)
        def _():
            read_win(j + 1, 1 - b)

        pltpu.make_async_copy(
            t_hbm.at[:, pl.ds(0, _W)], wbuf.at[b], rsem).wait()
        write_tiles(j, b)
        return carry

    lax.fori_loop(0, nw, step, 0)
    drain_writes(lax.rem(nw - 1, 2))


def _unpack_body(ct_hbm, pt_hbm, c_out, p_out, wbuf, rsem, wsem):
    cid = lax.axis_index("c")
    sid = lax.axis_index("s")
    wid = sid * _NC + cid
    nw = 61 + jnp.where(wid < _NWIN - 61 * _NW, 1, 0)
    _unpack_one(ct_hbm, c_out, wbuf, rsem, wsem, wid, nw)
    _unpack_one(pt_hbm, p_out, wbuf, rsem, wsem, wid, nw)

    def tail(t_hbm, out4):
        c0t = pl.multiple_of(wid * 0 + (_NT - 1) * _CHUNK, _CHUNK)
        pltpu.sync_copy(t_hbm.at[:, pl.ds(c0t, _CHUNK)],
                        wbuf.at[0, :, pl.ds(0, _CHUNK)])
        for kb in range(_KB):
            pltpu.sync_copy(
                wbuf.at[0, pl.ds(kb * 8, 8), pl.ds(0, _CHUNK)],
                out4.at[kb, _NT - 1])

    @pl.when(wid == _NW - 1)
    def _():
        tail(ct_hbm, c_out)

    @pl.when(wid == _NW - 2)
    def _():
        tail(pt_hbm, p_out)


_unpack_call = functools.partial(
    pl.kernel,
    mesh=plsc.VectorSubcoreMesh(core_axis_name="c", subcore_axis_name="s"),
    out_type=(jax.ShapeDtypeStruct((_KB, _NT, 8, _CHUNK), jnp.float32),
              jax.ShapeDtypeStruct((_KB, _NT, 8, _CHUNK), jnp.float32)),
    compiler_params=pltpu.CompilerParams(
        needs_layout_passes=False, use_tc_tiling_on_sc=True,
        disable_bounds_checks=True),
    scratch_types=[
        pltpu.VMEM((2, N_FACT, _W), jnp.float32),
        pltpu.SemaphoreType.DMA,
        pltpu.SemaphoreType.DMA,
    ],
)(_unpack_body)


def _gather_body(x0_hbm, x1_hbm, c_flat, p_flat, wb_hbm, out_hbm,
                 idx_c, idx_p, pos_c, pos_p, c_cols, p_cols, wb_v, out_v,
                 sem):
    cid = lax.axis_index("c")
    sid = lax.axis_index("s")
    wid = sid * _NC + cid

    pltpu.sync_copy(x0_hbm.at[pl.ds(wid * _NCH, _NCH)], idx_c)
    pltpu.sync_copy(x1_hbm.at[pl.ds(wid * _NCH, _NCH)], idx_p)
    pltpu.sync_copy(wb_hbm, wb_v)

    def to_pos(j, carry):
        for src, dst in ((idx_c, pos_c), (idx_p, pos_p)):
            for g in range(_CHUNK // LANES):
                iv = src[j, pl.ds(g * LANES, LANES)]
                dst[j, pl.ds(g * LANES, LANES)] = (
                    (iv >> 7) * 1024 + (iv & 127))
        return carry

    lax.fori_loop(0, _NCH, to_pos, 0)

    for k0 in range(0, N_FACT, 2):
        cps = []
        for k in (k0, k0 + 1):
            base = ((k // 8) * _NT * 1024) + (k % 8) * _CHUNK
            span = _FLAT - base
            for j in range(_NCH):
                cps.append(pltpu.async_copy(
                    c_flat.at[pl.ds(base, span)].at[pos_c.at[j]],
                    c_cols.at[k, pl.ds(j * _CHUNK, _CHUNK)], sem))
                cps.append(pltpu.async_copy(
                    p_flat.at[pl.ds(base, span)].at[pos_p.at[j]],
                    p_cols.at[k, pl.ds(j * _CHUNK, _CHUNK)], sem))
        for cp in cps:
            cp.wait()

    w = wb_v[0, :]
    b = wb_v[1, :]

    def grp(g, carry):
        acc = jnp.zeros((LANES,), jnp.float32)
        for k in range(N_FACT):
            cv = c_cols[k, pl.ds(g * LANES, LANES)]
            pv = p_cols[k, pl.ds(g * LANES, LANES)]
            acc = acc + cv * pv
        y = acc * w + b
        out_v[pl.ds(g * LANES, LANES)] = 1.0 / (1.0 + jnp.exp(-y))
        return carry

    lax.fori_loop(0, _BPW // LANES, grp, 0)
    pltpu.sync_copy(out_v, out_hbm.at[pl.ds(wid * _BPW, _BPW)])


_gather_call = functools.partial(
    pl.kernel,
    mesh=plsc.VectorSubcoreMesh(core_axis_name="c", subcore_axis_name="s"),
    out_type=jax.ShapeDtypeStruct((N_ROWS,), jnp.float32),
    compiler_params=pltpu.CompilerParams(
        needs_layout_passes=False, use_tc_tiling_on_sc=False),
    scratch_types=[
        pltpu.VMEM((_NCH, _CHUNK), jnp.int32),
        pltpu.VMEM((_NCH, _CHUNK), jnp.int32),
        pltpu.VMEM((_NCH, _CHUNK), jnp.int32),
        pltpu.VMEM((_NCH, _CHUNK), jnp.int32),
        pltpu.VMEM((N_FACT, _BPW), jnp.float32),
        pltpu.VMEM((N_FACT, _BPW), jnp.float32),
        pltpu.VMEM((2, LANES), jnp.float32),
        pltpu.VMEM((_BPW,), jnp.float32),
        pltpu.SemaphoreType.DMA,
    ],
)(_gather_body)


def kernel(x, cust_embedding, prod_embedding, out_w, out_b):
    x0 = x[:, 0].reshape(_IDX_ROWS, _CHUNK)
    x1 = x[:, 1].reshape(_IDX_ROWS, _CHUNK)
    wb = jnp.stack([
        jnp.full((LANES,), out_w[0, 0], jnp.float32),
        jnp.full((LANES,), out_b[0], jnp.float32),
    ])
    c4, p4 = _unpack_call(cust_embedding.T, prod_embedding.T)
    c_flat = c4.reshape(_FLAT)
    p_flat = p4.reshape(_FLAT)
    out = _gather_call(x0, x1, c_flat, p_flat, wb)
    return out.reshape(N_ROWS, 1)


# speedup vs baseline: 19.9399x; 1.0005x over previous
import functools

import jax
import jax.numpy as jnp
from jax import lax
from jax.experimental import pallas as pl
from jax.experimental.pallas import tpu as pltpu
from jax.experimental.pallas import tpu_sc as plsc

N_ROWS = 16384
N_FACT = 32
LANES = 16
VOCAB = 1000000

_info = plsc.get_sparse_core_info()
_NC, _NS = _info.num_cores, _info.num_subcores
_NW = _NC * _NS
_BPW = N_ROWS // _NW
_CHUNK = 128
_NCH = _BPW // _CHUNK
_IDX_ROWS = N_ROWS // _CHUNK

_KB = N_FACT // 8
_NT = (VOCAB + _CHUNK - 1) // _CHUNK
_W = 768
_TPW = _W // _CHUNK
_NWIN = (_NT - 1) // _TPW
_FLAT = _KB * _NT * 8 * _CHUNK


def _unpack_one(t_hbm, out4, wbuf, rsem, wsem, wid, nw):
    def read_win(j, b):
        c0 = pl.multiple_of((j * _NW + wid) * _W, _W)
        return pltpu.async_copy(t_hbm.at[:, pl.ds(c0, _W)], wbuf.at[b], rsem)

    def write_tiles(j, b):
        t0 = (j * _NW + wid) * _TPW
        for kb in range(_KB):
            for c in range(_TPW):
                pltpu.async_copy(
                    wbuf.at[b, pl.ds(kb * 8, 8), pl.ds(c * _CHUNK, _CHUNK)],
                    out4.at[kb, t0 + c], wsem)

    def drain_writes(b):
        for _ in range(_KB * _TPW):
            pltpu.make_async_copy(
                wbuf.at[b, pl.ds(0, 8), pl.ds(0, _CHUNK)],
                out4.at[0, 0], wsem).wait()

    read_win(0, 0)

    def step(j, carry):
        b = lax.rem(j, 2)

        @pl.when(j >= 1)
        def _():
            drain_writes(1 - b)

        @pl.when(j + 1 < nw)
        def _():
            read_win(j + 1, 1 - b)

        pltpu.make_async_copy(
            t_hbm.at[:, pl.ds(0, _W)], wbuf.at[b], rsem).wait()
        write_tiles(j, b)
        return carry

    lax.fori_loop(0, nw, step, 0)
    drain_writes(lax.rem(nw - 1, 2))


def _unpack_body(ct_hbm, pt_hbm, c_out, p_out, wbuf, rsem, wsem):
    cid = lax.axis_index("c")
    sid = lax.axis_index("s")
    wid = sid * _NC + cid
    base_nw = _NWIN // _NW
    nw = base_nw + jnp.where(wid < _NWIN - base_nw * _NW, 1, 0)
    _unpack_one(ct_hbm, c_out, wbuf, rsem, wsem, wid, nw)
    _unpack_one(pt_hbm, p_out, wbuf, rsem, wsem, wid, nw)

    def tail(t_hbm, out4):
        c0t = pl.multiple_of(wid * 0 + (_NT - 1) * _CHUNK, _CHUNK)
        pltpu.sync_copy(t_hbm.at[:, pl.ds(c0t, _CHUNK)],
                        wbuf.at[0, :, pl.ds(0, _CHUNK)])
        for kb in range(_KB):
            pltpu.sync_copy(
                wbuf.at[0, pl.ds(kb * 8, 8), pl.ds(0, _CHUNK)],
                out4.at[kb, _NT - 1])

    @pl.when(wid == _NW - 1)
    def _():
        tail(ct_hbm, c_out)

    @pl.when(wid == _NW - 2)
    def _():
        tail(pt_hbm, p_out)


_unpack_call = functools.partial(
    pl.kernel,
    mesh=plsc.VectorSubcoreMesh(core_axis_name="c", subcore_axis_name="s"),
    out_type=(jax.ShapeDtypeStruct((_KB, _NT, 8, _CHUNK), jnp.float32),
              jax.ShapeDtypeStruct((_KB, _NT, 8, _CHUNK), jnp.float32)),
    compiler_params=pltpu.CompilerParams(
        needs_layout_passes=False, use_tc_tiling_on_sc=True,
        disable_bounds_checks=True),
    scratch_types=[
        pltpu.VMEM((2, N_FACT, _W), jnp.float32),
        pltpu.SemaphoreType.DMA,
        pltpu.SemaphoreType.DMA,
    ],
)(_unpack_body)


def _gather_body(x0_hbm, x1_hbm, c_flat, p_flat, wb_hbm, out_hbm,
                 idx_c, idx_p, pos_c, pos_p, c_cols, p_cols, wb_v, out_v,
                 sem):
    cid = lax.axis_index("c")
    sid = lax.axis_index("s")
    wid = sid * _NC + cid

    pltpu.sync_copy(x0_hbm.at[pl.ds(wid * _NCH, _NCH)], idx_c)
    pltpu.sync_copy(x1_hbm.at[pl.ds(wid * _NCH, _NCH)], idx_p)
    pltpu.sync_copy(wb_hbm, wb_v)

    def to_pos(j, carry):
        for src, dst in ((idx_c, pos_c), (idx_p, pos_p)):
            for g in range(_CHUNK // LANES):
                iv = src[j, pl.ds(g * LANES, LANES)]
                dst[j, pl.ds(g * LANES, LANES)] = (
                    (iv >> 7) * 1024 + (iv & 127))
        return carry

    lax.fori_loop(0, _NCH, to_pos, 0)

    for k0 in range(0, N_FACT, 2):
        cps = []
        for k in (k0, k0 + 1):
            base = ((k // 8) * _NT * 1024) + (k % 8) * _CHUNK
            span = _FLAT - base
            for j in range(_NCH):
                cps.append(pltpu.async_copy(
                    c_flat.at[pl.ds(base, span)].at[pos_c.at[j]],
                    c_cols.at[k, pl.ds(j * _CHUNK, _CHUNK)], sem))
                cps.append(pltpu.async_copy(
                    p_flat.at[pl.ds(base, span)].at[pos_p.at[j]],
                    p_cols.at[k, pl.ds(j * _CHUNK, _CHUNK)], sem))
        for cp in cps:
            cp.wait()

    w = wb_v[0, :]
    b = wb_v[1, :]

    def grp(g, carry):
        acc = jnp.zeros((LANES,), jnp.float32)
        for k in range(N_FACT):
            cv = c_cols[k, pl.ds(g * LANES, LANES)]
            pv = p_cols[k, pl.ds(g * LANES, LANES)]
            acc = acc + cv * pv
        y = acc * w + b
        out_v[pl.ds(g * LANES, LANES)] = 1.0 / (1.0 + jnp.exp(-y))
        return carry

    lax.fori_loop(0, _BPW // LANES, grp, 0)
    pltpu.sync_copy(out_v, out_hbm.at[pl.ds(wid * _BPW, _BPW)])


_gather_call = functools.partial(
    pl.kernel,
    mesh=plsc.VectorSubcoreMesh(core_axis_name="c", subcore_axis_name="s"),
    out_type=jax.ShapeDtypeStruct((N_ROWS,), jnp.float32),
    compiler_params=pltpu.CompilerParams(
        needs_layout_passes=False, use_tc_tiling_on_sc=False),
    scratch_types=[
        pltpu.VMEM((_NCH, _CHUNK), jnp.int32),
        pltpu.VMEM((_NCH, _CHUNK), jnp.int32),
        pltpu.VMEM((_NCH, _CHUNK), jnp.int32),
        pltpu.VMEM((_NCH, _CHUNK), jnp.int32),
        pltpu.VMEM((N_FACT, _BPW), jnp.float32),
        pltpu.VMEM((N_FACT, _BPW), jnp.float32),
        pltpu.VMEM((2, LANES), jnp.float32),
        pltpu.VMEM((_BPW,), jnp.float32),
        pltpu.SemaphoreType.DMA,
    ],
)(_gather_body)


def kernel(x, cust_embedding, prod_embedding, out_w, out_b):
    x0 = x[:, 0].reshape(_IDX_ROWS, _CHUNK)
    x1 = x[:, 1].reshape(_IDX_ROWS, _CHUNK)
    wb = jnp.stack([
        jnp.full((LANES,), out_w[0, 0], jnp.float32),
        jnp.full((LANES,), out_b[0], jnp.float32),
    ])
    c4, p4 = _unpack_call(cust_embedding.T, prod_embedding.T)
    c_flat = c4.reshape(_FLAT)
    p_flat = p4.reshape(_FLAT)
    out = _gather_call(x0, x1, c_flat, p_flat, wb)
    return out.reshape(N_ROWS, 1)
